# Initial kernel scaffold; baseline (speedup 1.0000x reference)
import jax
import jax.numpy as jnp
from jax.experimental import pallas as pl


def kernel(inx, th, params):
    raise NotImplementedError("write your pallas kernel here")


# speedup vs baseline: 20.8461x; 20.8461x over previous
import jax
import jax.numpy as jnp
import numpy as np
from jax.experimental import pallas as pl
from jax.experimental.pallas import tpu as pltpu

H = 512
W = 512
HALO = 8
TILE = 128
EXT = TILE + 2 * HALO
L = EXT * W
NT = H // TILE
LPAD = (H + 2 * HALO) * W

TAPS5 = [(dy, dx) for dy in range(-2, 3) for dx in range(-2, 3)]
TAPS3 = [(dy, dx) for dy in range(-1, 2) for dx in range(-1, 2)]
PHASES = [(0, 0), (0, 1), (1, 0), (1, 1)]


def _t_range(r):
    return range(-2, 3) if r == 0 else range(-1, 3)


def _net_kernel(xf_ref, th_ref, *refs):
    (wfirst, bfirst, wfl2, bfl2,
     wred, bred, wrl2, brl2,
     wm1, bm1, wml21, bml21,
     wm2, bm2, wml22, bml22,
     wm3, bm3, wml23, bml23,
     wm4, bm4, wml24, bml24,
     wexp, bexp, wel2, bel2,
     wll1, bll1, wlast, blast, out_ref) = refs

    t = pl.program_id(1)
    th = th_ref[0, 0]

    colmask_cache = {}

    def colmask(dx, dtype):
        key = (dx, dtype)
        if key not in colmask_cache:
            c = jax.lax.broadcasted_iota(jnp.int32, (1, L), 1) & (W - 1)
            if dx > 0:
                v = c < (W - dx)
            else:
                v = c >= (-dx)
            colmask_cache[key] = v.astype(dtype)
        return colmask_cache[key]

    def shift(x, dy, dx):
        cn, ln = x.shape
        off = dy * W + dx
        if off > 0:
            y = jnp.concatenate(
                [x[:, off:], jnp.zeros((cn, off), x.dtype)], axis=1)
        elif off < 0:
            y = jnp.concatenate(
                [jnp.zeros((cn, -off), x.dtype), x[:, :off]], axis=1)
        else:
            y = x
        if dx != 0:
            y = y * colmask(dx, x.dtype)
        return y

    def conv_mm(x, taps, wp, chunk):
        cn = x.shape[0]
        acc = None
        for i0 in range(0, len(taps), chunk):
            sub = taps[i0:i0 + chunk]
            if len(sub) == 1 and sub[0] == (0, 0):
                op = x
            else:
                op = jnp.concatenate(
                    [shift(x, dy, dx) for dy, dx in sub], axis=0)
            wsub = wp[:, i0 * cn:(i0 + len(sub)) * cn]
            part = jax.lax.dot_general(
                wsub, op, (((1,), (0,)), ((), ())),
                preferred_element_type=jnp.float32)
            acc = part if acc is None else acc + part
        return acc

    x1 = xf_ref[0, 0:1, pl.ds(t * (TILE * W), L)]

    row = jax.lax.shift_right_logical(
        jax.lax.broadcasted_iota(jnp.int32, (1, L), 1), 9)
    absrow = t * TILE - HALO + row
    p = ((absrow >= 0) & (absrow < H)).astype(jnp.float32)

    s = None
    cnt = None
    for dy, dx in TAPS3:
        s = shift(x1, dy, dx) if s is None else s + shift(x1, dy, dx)
        cnt = shift(p, dy, dx) if cnt is None else cnt + shift(p, dy, dx)
    blur = s / jnp.maximum(cnt, 1.0)
    loss = jnp.abs(x1 - blur)
    mask0 = (loss >= th).astype(jnp.float32)
    mask = None
    for dy, dx in TAPS3:
        sm = shift(mask0, dy, dx)
        mask = sm if mask is None else jnp.maximum(mask, sm)
    inv = 1.0 - mask

    def block(x, taps, wp, bp, wl2_, bl2_, chunk):
        acc = conv_mm(x, taps, wp[:], chunk)
        acc = acc + bp[:]
        mh = wp.shape[0] - wl2_.shape[1]
        x1h = acc[:mh] * mask
        xl = (acc[mh:] * inv).astype(jnp.bfloat16)
        xl2 = (jax.lax.dot_general(
            wl2_[:], xl, (((1,), (0,)), ((), ())),
            preferred_element_type=jnp.float32) + bl2_[:]) * inv
        return (jnp.maximum(x1h + xl2, 0.0) * p).astype(jnp.bfloat16)

    x = block(x1.astype(jnp.bfloat16), TAPS5, wfirst, bfirst,
              wfl2, bfl2, 25)
    x = block(x, [(0, 0)], wred, bred, wrl2, brl2, 1)
    x = block(x, TAPS3, wm1, bm1, wml21, bml21, 3)
    x = block(x, TAPS3, wm2, bm2, wml22, bml22, 3)
    x = block(x, TAPS3, wm3, bm3, wml23, bml23, 3)
    x = block(x, TAPS3, wm4, bm4, wml24, bml24, 3)
    x = block(x, [(0, 0)], wexp, bexp, wel2, bel2, 1)

    x2a = (jax.lax.dot_general(
        wll1[:], x, (((1,), (0,)), ((), ())),
        preferred_element_type=jnp.float32) + bll1[:]) * p
    xz = jnp.concatenate([x, x2a.astype(jnp.bfloat16)], axis=0)
    wl = wlast[:]
    bh_l = blast[0, 0]
    bl2_l = blast[1, 0]

    lo = HALO * W
    ys = []
    for (ry, rx) in PHASES:
        m = ry * 2 + rx
        acc = None
        for tyi, ty in enumerate(range(-2, 3)):
            if ty not in _t_range(ry):
                continue
            xzy = shift(xz, ty, 0)
            for txi, tx in enumerate(range(-2, 3)):
                if tx not in _t_range(rx):
                    continue
                xs = shift(xzy, 0, tx)
                col = m * 25 + tyi * 5 + txi
                contrib = wl[:, col:col + 1] * xs
                acc = contrib if acc is None else acc + contrib
        s1 = jnp.sum(acc[:32], axis=0, keepdims=True)
        s2 = jnp.sum(acc[32:], axis=0, keepdims=True)
        ym = (s1 + bh_l) * mask + (s2 + bl2_l) * inv
        ys.append(ym[:, lo:lo + TILE * W])
    out_ref[0, 0] = jnp.concatenate(ys, axis=0)


def _pack_weights(params):
    bf = lambda a: a.astype(jnp.bfloat16)
    r4 = lambda a: a.reshape(a.shape[0], -1)

    def pack_block(p):
        hw = p['hw'].transpose(0, 2, 3, 1).reshape(p['hw'].shape[0], -1)
        l1 = p['l1w'].transpose(0, 2, 3, 1).reshape(p['l1w'].shape[0], -1)
        wp = bf(jnp.concatenate([hw, l1], axis=0))
        bp = jnp.concatenate([p['hb'], p['l1b']])[:, None]
        return wp, bp, bf(r4(p['l2w'])), p['l2b'][:, None]

    out = []
    out += list(pack_block(params['first']))
    out += list(pack_block(params['reduction']))
    for nm in ('mid1', 'mid2', 'mid3', 'mid4'):
        out += list(pack_block(params[nm]))
    out += list(pack_block(params['expansion']))

    lp = params['last']
    out.append(bf(r4(lp['l1w'])))
    out.append(lp['l1b'][:, None])

    iy = np.zeros((4, 5, 5), np.int32)
    ix = np.zeros((4, 5, 5), np.int32)
    valid = np.zeros((4, 5, 5), np.float32)
    for (ry, rx) in PHASES:
        m = ry * 2 + rx
        for tyi, ty in enumerate(range(-2, 3)):
            for txi, tx in enumerate(range(-2, 3)):
                if ty in _t_range(ry) and tx in _t_range(rx):
                    ay = 4 + 2 * ty if ry == 0 else 3 + 2 * ty
                    ax = 4 + 2 * tx if rx == 0 else 3 + 2 * tx
                    iy[m, tyi, txi] = 8 - ay
                    ix[m, tyi, txi] = 8 - ax
                    valid[m, tyi, txi] = 1.0
    vg = jnp.asarray(valid)
    whw = lp['hw'][:, 0][:, iy, ix] * vg
    wl2 = lp['l2w'][:, 0][:, iy, ix] * vg
    wlast = jnp.concatenate(
        [whw.reshape(32, 100), wl2.reshape(8, 100)], axis=0)
    out.append(wlast)
    out.append(jnp.stack([lp['hb'][0], lp['l2b'][0]])[:, None])
    return out


def kernel(inx, th, params):
    n = inx.shape[0]
    xpad = jnp.pad(inx[:, 0], ((0, 0), (HALO, HALO), (0, 0)))
    xflat = xpad.reshape(n, 1, LPAD)
    th2 = th.reshape(1, 1)
    ws = _pack_weights(params)

    full = lambda a: pl.BlockSpec(a.shape, lambda i, j: (0,) * a.ndim)
    in_specs = [
        pl.BlockSpec((1, 1, LPAD), lambda i, j: (i, 0, 0)),
        pl.BlockSpec(memory_space=pltpu.SMEM),
    ] + [full(a) for a in ws[:-1]] + [pl.BlockSpec(memory_space=pltpu.SMEM)]

    out = pl.pallas_call(
        _net_kernel,
        grid=(n, NT),
        in_specs=in_specs,
        out_specs=pl.BlockSpec((1, 1, 4, TILE * W), lambda i, j: (i, j, 0, 0)),
        out_shape=jax.ShapeDtypeStruct((n, NT, 4, TILE * W), jnp.float32),
    )(xflat, th2, *ws)

    y = out.reshape(n, NT, 2, 2, TILE, W)
    y = y.transpose(0, 1, 4, 2, 5, 3).reshape(n, 1, 2 * H, 2 * W)
    return y


# speedup vs baseline: 21.1393x; 1.0141x over previous
import jax
import jax.numpy as jnp
import numpy as np
from jax.experimental import pallas as pl
from jax.experimental.pallas import tpu as pltpu

H = 512
W = 512
HALO = 8
TILE = 128
EXT = TILE + 2 * HALO
L = EXT * W
NT = H // TILE
LPAD = (H + 2 * HALO) * W

TAPS5 = [(dy, dx) for dy in range(-2, 3) for dx in range(-2, 3)]
TAPS3 = [(dy, dx) for dy in range(-1, 2) for dx in range(-1, 2)]
PHASES = [(0, 0), (0, 1), (1, 0), (1, 1)]


def _t_range(r):
    return range(-2, 3) if r == 0 else range(-1, 3)


def _net_kernel(xf_ref, th_ref, *refs):
    (wfirst, bfirst, wfl2, bfl2,
     wred, bred, wrl2, brl2,
     wm1, bm1, wml21, bml21,
     wm2, bm2, wml22, bml22,
     wm3, bm3, wml23, bml23,
     wm4, bm4, wml24, bml24,
     wexp, bexp, wel2, bel2,
     wll1, bll1, wlast, blast, out_ref) = refs

    t = pl.program_id(1)
    th = th_ref[0, 0]

    colmask_cache = {}

    def colmask(dx, dtype):
        key = (dx, dtype)
        if key not in colmask_cache:
            c = jax.lax.broadcasted_iota(jnp.int32, (1, L), 1) & (W - 1)
            if dx > 0:
                v = c < (W - dx)
            else:
                v = c >= (-dx)
            colmask_cache[key] = v.astype(dtype)
        return colmask_cache[key]

    def shift(x, dy, dx):
        cn, ln = x.shape
        off = dy * W + dx
        if off > 0:
            y = jnp.concatenate(
                [x[:, off:], jnp.zeros((cn, off), x.dtype)], axis=1)
        elif off < 0:
            y = jnp.concatenate(
                [jnp.zeros((cn, -off), x.dtype), x[:, :off]], axis=1)
        else:
            y = x
        if dx != 0:
            y = y * colmask(dx, x.dtype)
        return y

    def conv_mm(x, taps, wp, chunk):
        cn = x.shape[0]
        acc = None
        for i0 in range(0, len(taps), chunk):
            sub = taps[i0:i0 + chunk]
            if len(sub) == 1 and sub[0] == (0, 0):
                op = x
            else:
                op = jnp.concatenate(
                    [shift(x, dy, dx) for dy, dx in sub], axis=0)
            wsub = wp[:, i0 * cn:(i0 + len(sub)) * cn]
            part = jax.lax.dot_general(
                wsub, op, (((1,), (0,)), ((), ())),
                preferred_element_type=jnp.float32)
            acc = part if acc is None else acc + part
        return acc

    def conv3x3_mm(x, wp):
        cn = x.shape[0]
        op0 = jnp.concatenate(
            [shift(x, 0, dx) for dx in (-1, 0, 1)], axis=0)
        op = jnp.concatenate(
            [shift(op0, dy, 0) for dy in (-1, 0, 1)], axis=0)
        return jax.lax.dot_general(
            wp, op, (((1,), (0,)), ((), ())),
            preferred_element_type=jnp.float32)

    x1 = xf_ref[0, 0:1, pl.ds(t * (TILE * W), L)]

    row = jax.lax.shift_right_logical(
        jax.lax.broadcasted_iota(jnp.int32, (1, L), 1), 9)
    absrow = t * TILE - HALO + row
    p = ((absrow >= 0) & (absrow < H)).astype(jnp.float32)

    s = None
    cnt = None
    for dy, dx in TAPS3:
        s = shift(x1, dy, dx) if s is None else s + shift(x1, dy, dx)
        cnt = shift(p, dy, dx) if cnt is None else cnt + shift(p, dy, dx)
    blur = s / jnp.maximum(cnt, 1.0)
    loss = jnp.abs(x1 - blur)
    mask0 = (loss >= th).astype(jnp.float32)
    mask = None
    for dy, dx in TAPS3:
        sm = shift(mask0, dy, dx)
        mask = sm if mask is None else jnp.maximum(mask, sm)
    inv = 1.0 - mask

    def block(x, taps, wp, bp, wl2_, bl2_, chunk):
        if taps is TAPS3:
            acc = conv3x3_mm(x, wp[:])
        else:
            acc = conv_mm(x, taps, wp[:], chunk)
        acc = acc + bp[:]
        mh = wp.shape[0] - wl2_.shape[1]
        x1h = acc[:mh] * mask
        xl = (acc[mh:] * inv).astype(jnp.bfloat16)
        xl2 = (jax.lax.dot_general(
            wl2_[:], xl, (((1,), (0,)), ((), ())),
            preferred_element_type=jnp.float32) + bl2_[:]) * inv
        return (jnp.maximum(x1h + xl2, 0.0) * p).astype(jnp.bfloat16)

    x = block(x1.astype(jnp.bfloat16), TAPS5, wfirst, bfirst,
              wfl2, bfl2, 25)
    x = block(x, [(0, 0)], wred, bred, wrl2, brl2, 1)
    x = block(x, TAPS3, wm1, bm1, wml21, bml21, 3)
    x = block(x, TAPS3, wm2, bm2, wml22, bml22, 3)
    x = block(x, TAPS3, wm3, bm3, wml23, bml23, 3)
    x = block(x, TAPS3, wm4, bm4, wml24, bml24, 3)
    x = block(x, [(0, 0)], wexp, bexp, wel2, bel2, 1)

    x2a = (jax.lax.dot_general(
        wll1[:], x, (((1,), (0,)), ((), ())),
        preferred_element_type=jnp.float32) + bll1[:]) * p
    xz = jnp.concatenate([x, x2a.astype(jnp.bfloat16)], axis=0)
    wl = wlast[:]
    bh_l = blast[0, 0]
    bl2_l = blast[1, 0]

    lo = HALO * W
    ys = []
    for (ry, rx) in PHASES:
        m = ry * 2 + rx
        acc = None
        for tyi, ty in enumerate(range(-2, 3)):
            if ty not in _t_range(ry):
                continue
            xzy = shift(xz, ty, 0)
            for txi, tx in enumerate(range(-2, 3)):
                if tx not in _t_range(rx):
                    continue
                xs = shift(xzy, 0, tx)
                col = m * 25 + tyi * 5 + txi
                contrib = wl[:, col:col + 1] * xs
                acc = contrib if acc is None else acc + contrib
        s1 = jnp.sum(acc[:32], axis=0, keepdims=True)
        s2 = jnp.sum(acc[32:], axis=0, keepdims=True)
        ym = (s1 + bh_l) * mask + (s2 + bl2_l) * inv
        ys.append(ym[:, lo:lo + TILE * W])
    out_ref[0, 0] = jnp.concatenate(ys, axis=0)


def _pack_weights(params):
    bf = lambda a: a.astype(jnp.bfloat16)
    r4 = lambda a: a.reshape(a.shape[0], -1)

    def pack_block(p):
        hw = p['hw'].transpose(0, 2, 3, 1).reshape(p['hw'].shape[0], -1)
        l1 = p['l1w'].transpose(0, 2, 3, 1).reshape(p['l1w'].shape[0], -1)
        wp = bf(jnp.concatenate([hw, l1], axis=0))
        bp = jnp.concatenate([p['hb'], p['l1b']])[:, None]
        return wp, bp, bf(r4(p['l2w'])), p['l2b'][:, None]

    out = []
    out += list(pack_block(params['first']))
    out += list(pack_block(params['reduction']))
    for nm in ('mid1', 'mid2', 'mid3', 'mid4'):
        out += list(pack_block(params[nm]))
    out += list(pack_block(params['expansion']))

    lp = params['last']
    out.append(bf(r4(lp['l1w'])))
    out.append(lp['l1b'][:, None])

    iy = np.zeros((4, 5, 5), np.int32)
    ix = np.zeros((4, 5, 5), np.int32)
    valid = np.zeros((4, 5, 5), np.float32)
    for (ry, rx) in PHASES:
        m = ry * 2 + rx
        for tyi, ty in enumerate(range(-2, 3)):
            for txi, tx in enumerate(range(-2, 3)):
                if ty in _t_range(ry) and tx in _t_range(rx):
                    ay = 4 + 2 * ty if ry == 0 else 3 + 2 * ty
                    ax = 4 + 2 * tx if rx == 0 else 3 + 2 * tx
                    iy[m, tyi, txi] = 8 - ay
                    ix[m, tyi, txi] = 8 - ax
                    valid[m, tyi, txi] = 1.0
    vg = jnp.asarray(valid)
    whw = lp['hw'][:, 0][:, iy, ix] * vg
    wl2 = lp['l2w'][:, 0][:, iy, ix] * vg
    wlast = jnp.concatenate(
        [whw.reshape(32, 100), wl2.reshape(8, 100)], axis=0)
    out.append(wlast)
    out.append(jnp.stack([lp['hb'][0], lp['l2b'][0]])[:, None])
    return out


def kernel(inx, th, params):
    n = inx.shape[0]
    xpad = jnp.pad(inx[:, 0], ((0, 0), (HALO, HALO), (0, 0)))
    xflat = xpad.reshape(n, 1, LPAD)
    th2 = th.reshape(1, 1)
    ws = _pack_weights(params)

    full = lambda a: pl.BlockSpec(a.shape, lambda i, j: (0,) * a.ndim)
    in_specs = [
        pl.BlockSpec((1, 1, LPAD), lambda i, j: (i, 0, 0)),
        pl.BlockSpec(memory_space=pltpu.SMEM),
    ] + [full(a) for a in ws[:-1]] + [pl.BlockSpec(memory_space=pltpu.SMEM)]

    out = pl.pallas_call(
        _net_kernel,
        grid=(n, NT),
        in_specs=in_specs,
        out_specs=pl.BlockSpec((1, 1, 4, TILE * W), lambda i, j: (i, j, 0, 0)),
        out_shape=jax.ShapeDtypeStruct((n, NT, 4, TILE * W), jnp.float32),
        compiler_params=pltpu.CompilerParams(
            dimension_semantics=("parallel", "arbitrary")),
    )(xflat, th2, *ws)

    y = out.reshape(n, NT, 2, 2, TILE, W)
    y = y.transpose(0, 1, 4, 2, 5, 3).reshape(n, 1, 2 * H, 2 * W)
    return y
